# baseline (speedup 1.0000x reference)
import jax
import jax.numpy as jnp
from jax import lax
from jax.experimental import pallas as pl
from jax.experimental.pallas import tpu as pltpu
from jax.experimental.pallas import tpu_sc as plsc

B = 16384
S = 200
D = 64
NC = 2
NS = 16
NW = NC * NS
RPW = B // NW
CR = 4
NCHUNK = RPW // CR
SPC = CR * S
NG = SPC // 16
CW = SPC * D

_DNUMS = lax.GatherDimensionNumbers(
    offset_dims=(), collapsed_slice_dims=(0,), start_index_map=(0,))


def _dg(v, idx):
    return lax.gather(v, idx[:, None], _DNUMS, slice_sizes=(1,),
                      mode=lax.GatherScatterMode.PROMISE_IN_BOUNDS)


def _body(x_hbm, table_hbm, out_hbm, tbuf, xb_a, xb_b, rv_a, rv_b,
          sx_a, sx_b, ss_a, ss_b, ss2_a, ss2_b):
    cid = lax.axis_index("c")
    sid = lax.axis_index("s")
    wid = sid * NC + cid
    tok_base = wid * RPW * S
    lane = lax.iota(jnp.int32, 16)
    zero16 = lane * 0

    pltpu.sync_copy(table_hbm, tbuf)
    t0 = [tbuf[pl.ds(k * 16, 16)] for k in range(4)]
    t1 = [tbuf[pl.ds(64 + k * 16, 16)] for k in range(4)]

    def x_slice(c):
        off = pl.multiple_of(tok_base + c * SPC, 32)
        return x_hbm.at[pl.ds(off, SPC)]

    def out_slice(c, h):
        off = pl.multiple_of((tok_base + c * SPC) * D + h * (CW // 2), 512)
        return out_hbm.at[pl.ds(off, CW // 2)]

    def compute_chunk(xb, rv):
        def group(g, carry):
            v = xb[pl.ds(g * 16, 16)]
            scan = plsc.cumsum(v)
            t = scan + carry
            ex = t - v
            gm = g % 25
            lam = jnp.where(gm == 0, 0, jnp.where(gm == 12, 8, 16))
            sub = _dg(ex, zero16 + jnp.minimum(lam, 15))
            tot = jnp.where(lane < lam, t, t - sub)
            par = tot & 1
            for l in range(16):
                msk = _dg(par, zero16 + l) != 0
                base = g * (16 * D) + l * D
                for k in range(4):
                    rv[pl.ds(base + k * 16, 16)] = jnp.where(msk, t1[k], t0[k])
            return _dg(tot, zero16 + 15)

        lax.fori_loop(0, NG, group, jnp.zeros((16,), jnp.int32), unroll=2)

    pltpu.async_copy(x_slice(0), xb_a, sx_a)
    pltpu.async_copy(x_slice(1), xb_b, sx_b)

    bufs = ((xb_a, rv_a, sx_a, ss_a, ss2_a), (xb_b, rv_b, sx_b, ss_b, ss2_b))

    def step(i, carry_unused):
        for j, (xb, rv, sx, ss, ss2) in enumerate(bufs):
            c = 2 * i + j
            rv_lo = rv.at[pl.ds(0, CW // 2)]
            rv_hi = rv.at[pl.ds(CW // 2, CW // 2)]
            pltpu.make_async_copy(x_slice(0), xb, sx).wait()

            @pl.when(i > 0)
            def _():
                pltpu.make_async_copy(rv_lo, out_slice(0, 0), ss).wait()
                pltpu.make_async_copy(rv_hi, out_slice(0, 1), ss2).wait()

            compute_chunk(xb, rv)
            pltpu.async_copy(rv_lo, out_slice(c, 0), ss)
            pltpu.async_copy(rv_hi, out_slice(c, 1), ss2)

            @pl.when(c + 2 < NCHUNK)
            def _():
                pltpu.async_copy(x_slice(c + 2), xb, sx)
        return carry_unused

    lax.fori_loop(0, NCHUNK // 2, step, 0)
    for rv, ss, ss2 in ((rv_a, ss_a, ss2_a), (rv_b, ss_b, ss2_b)):
        pltpu.make_async_copy(rv.at[pl.ds(0, CW // 2)], out_slice(0, 0), ss).wait()
        pltpu.make_async_copy(rv.at[pl.ds(CW // 2, CW // 2)], out_slice(0, 1), ss2).wait()


_sc_call = pl.kernel(
    _body,
    out_type=jax.ShapeDtypeStruct((B * S * D,), jnp.float32),
    mesh=plsc.VectorSubcoreMesh(core_axis_name="c", subcore_axis_name="s"),
    compiler_params=pltpu.CompilerParams(needs_layout_passes=False),
    scratch_types=[
        pltpu.VMEM((2 * D,), jnp.float32),
        pltpu.VMEM((SPC,), jnp.int32),
        pltpu.VMEM((SPC,), jnp.int32),
        pltpu.VMEM((CW,), jnp.float32),
        pltpu.VMEM((CW,), jnp.float32),
        pltpu.SemaphoreType.DMA,
        pltpu.SemaphoreType.DMA,
        pltpu.SemaphoreType.DMA,
        pltpu.SemaphoreType.DMA,
        pltpu.SemaphoreType.DMA,
        pltpu.SemaphoreType.DMA,
    ],
)


def kernel(x, table):
    out = _sc_call(x.reshape(-1), table.reshape(-1))
    return out.reshape(B, S, D)


# speedup vs baseline: 1.2556x; 1.2556x over previous
import jax
import jax.numpy as jnp
from jax import lax
from jax.experimental import pallas as pl
from jax.experimental.pallas import tpu as pltpu
from jax.experimental.pallas import tpu_sc as plsc

B = 16384
S = 200
D = 64
NC = 2
NS = 16
NW = NC * NS
RPW = B // NW
CR = 32
NCHUNK = RPW // CR
SPC = CR * S
NG = SPC // 16

_DNUMS = lax.GatherDimensionNumbers(
    offset_dims=(), collapsed_slice_dims=(0,), start_index_map=(0,))


def _dg(v, idx):
    return lax.gather(v, idx[:, None], _DNUMS, slice_sizes=(1,),
                      mode=lax.GatherScatterMode.PROMISE_IN_BOUNDS)


def _par_body(x_hbm, p_hbm, xb_a, xb_b, pv_a, pv_b,
              sx_a, sx_b, sp_a, sp_b):
    cid = lax.axis_index("c")
    sid = lax.axis_index("s")
    wid = sid * NC + cid
    tok_base = wid * RPW * S
    lane = lax.iota(jnp.int32, 16)
    zero16 = lane * 0

    def x_slice(c):
        off = pl.multiple_of(tok_base + c * SPC, 32)
        return x_hbm.at[pl.ds(off, SPC)]

    def p_slice(c):
        off = pl.multiple_of(tok_base + c * SPC, 32)
        return p_hbm.at[pl.ds(off, SPC)]

    def compute_chunk(xb, pv):
        def group(g, carry):
            v = xb[pl.ds(g * 16, 16)]
            scan = plsc.cumsum(v)
            t = scan + carry
            ex = t - v
            gm = g % 25
            lam = jnp.where(gm == 0, 0, jnp.where(gm == 12, 8, 16))
            sub = _dg(ex, zero16 + jnp.minimum(lam, 15))
            tot = jnp.where(lane < lam, t, t - sub)
            pv[pl.ds(g * 16, 16)] = (tot & 1).astype(jnp.float32)
            return _dg(tot, zero16 + 15)

        lax.fori_loop(0, NG, group, jnp.zeros((16,), jnp.int32), unroll=2)

    pltpu.async_copy(x_slice(0), xb_a, sx_a)
    pltpu.async_copy(x_slice(1), xb_b, sx_b)

    bufs = ((xb_a, pv_a, sx_a, sp_a), (xb_b, pv_b, sx_b, sp_b))

    def step(i, carry_unused):
        for j, (xb, pv, sx, sp) in enumerate(bufs):
            c = 2 * i + j
            pltpu.make_async_copy(x_slice(0), xb, sx).wait()

            @pl.when(i > 0)
            def _():
                pltpu.make_async_copy(pv, p_slice(0), sp).wait()

            compute_chunk(xb, pv)
            pltpu.async_copy(pv, p_slice(c), sp)

            @pl.when(c + 2 < NCHUNK)
            def _():
                pltpu.async_copy(x_slice(c + 2), xb, sx)
        return carry_unused

    lax.fori_loop(0, NCHUNK // 2, step, 0)
    pltpu.make_async_copy(pv_a, p_slice(0), sp_a).wait()
    pltpu.make_async_copy(pv_b, p_slice(0), sp_b).wait()


_sc_par = pl.kernel(
    _par_body,
    out_type=jax.ShapeDtypeStruct((B * S,), jnp.float32),
    mesh=plsc.VectorSubcoreMesh(core_axis_name="c", subcore_axis_name="s"),
    compiler_params=pltpu.CompilerParams(needs_layout_passes=False),
    scratch_types=[
        pltpu.VMEM((SPC,), jnp.int32),
        pltpu.VMEM((SPC,), jnp.int32),
        pltpu.VMEM((SPC,), jnp.float32),
        pltpu.VMEM((SPC,), jnp.float32),
        pltpu.SemaphoreType.DMA,
        pltpu.SemaphoreType.DMA,
        pltpu.SemaphoreType.DMA,
        pltpu.SemaphoreType.DMA,
    ],
)

BB = 64


def _tc_body(p_ref, tab_ref, o_ref):
    p = p_ref[...]
    t0 = tab_ref[0, :]
    diff = tab_ref[1, :] - t0
    o_ref[...] = p[:, :, None] * diff[None, None, :] + t0[None, None, :]


_tc_expand = pl.pallas_call(
    _tc_body,
    grid=(B // BB,),
    in_specs=[
        pl.BlockSpec((BB, S), lambda i: (i, 0)),
        pl.BlockSpec((2, D), lambda i: (0, 0)),
    ],
    out_specs=pl.BlockSpec((BB, S, D), lambda i: (i, 0, 0)),
    out_shape=jax.ShapeDtypeStruct((B, S, D), jnp.float32),
)


def kernel(x, table):
    p = _sc_par(x.reshape(-1)).reshape(B, S)
    return _tc_expand(p, table)


# speedup vs baseline: 1.4530x; 1.1572x over previous
import jax
import jax.numpy as jnp
from jax import lax
from jax.experimental import pallas as pl
from jax.experimental.pallas import tpu as pltpu
from jax.experimental.pallas import tpu_sc as plsc

B = 16384
S = 200
D = 64
H = S // 2
NC = 2
NS = 16
NW = NC * NS
RPW = B // NW
CR = 32
NCHUNK = RPW // CR
SPC = CR * S
NG = SPC // 16
HPC = SPC // 2

_DNUMS = lax.GatherDimensionNumbers(
    offset_dims=(), collapsed_slice_dims=(0,), start_index_map=(0,))


def _dg(v, idx):
    return lax.gather(v, idx[:, None], _DNUMS, slice_sizes=(1,),
                      mode=lax.GatherScatterMode.PROMISE_IN_BOUNDS)


def _par_body(x_hbm, pe_hbm, po_hbm, xb_a, xb_b, pe_a, po_a, pe_b, po_b,
              sx_a, sx_b, sp_a, sp_b):
    cid = lax.axis_index("c")
    sid = lax.axis_index("s")
    wid = sid * NC + cid
    tok_base = wid * RPW * S
    lane = lax.iota(jnp.int32, 16)
    zero16 = lane * 0
    even = (lane & 7) * 2
    odd = even + 1

    def x_slice(c):
        off = pl.multiple_of(tok_base + c * SPC, 32)
        return x_hbm.at[pl.ds(off, SPC)]

    def p_slice(hbm, c):
        off = pl.multiple_of((tok_base + c * SPC) // 2, 32)
        return hbm.at[pl.ds(off, HPC)]

    def compute_chunk(xb, pe, po):
        def scan_group(g, carry):
            v = xb[pl.ds(g * 16, 16)]
            scan = plsc.cumsum(v)
            t = scan + carry
            ex = t - v
            gm = g % 25
            lam = jnp.where(gm == 0, 0, jnp.where(gm == 12, 8, 16))
            sub = _dg(ex, zero16 + jnp.minimum(lam, 15))
            tot = jnp.where(lane < lam, t, t - sub)
            return tot & 1, _dg(tot, zero16 + 15)

        def gpair(g2, carry):
            pa, carry = scan_group(2 * g2, carry)
            pb, carry = scan_group(2 * g2 + 1, carry)
            ev = jnp.where(lane < 8, _dg(pa, even), _dg(pb, even))
            od = jnp.where(lane < 8, _dg(pa, odd), _dg(pb, odd))
            pe[pl.ds(g2 * 16, 16)] = ev.astype(jnp.float32)
            po[pl.ds(g2 * 16, 16)] = od.astype(jnp.float32)
            return carry

        lax.fori_loop(0, NG // 2, gpair, jnp.zeros((16,), jnp.int32))

    pltpu.async_copy(x_slice(0), xb_a, sx_a)
    pltpu.async_copy(x_slice(1), xb_b, sx_b)

    bufs = ((xb_a, pe_a, po_a, sx_a, sp_a), (xb_b, pe_b, po_b, sx_b, sp_b))

    def step(i, carry_unused):
        for j, (xb, pe, po, sx, sp) in enumerate(bufs):
            c = 2 * i + j
            pltpu.make_async_copy(x_slice(0), xb, sx).wait()

            @pl.when(i > 0)
            def _():
                pltpu.make_async_copy(pe, p_slice(pe_hbm, 0), sp).wait()
                pltpu.make_async_copy(po, p_slice(po_hbm, 0), sp).wait()

            compute_chunk(xb, pe, po)
            pltpu.async_copy(pe, p_slice(pe_hbm, c), sp)
            pltpu.async_copy(po, p_slice(po_hbm, c), sp)

            @pl.when(c + 2 < NCHUNK)
            def _():
                pltpu.async_copy(x_slice(c + 2), xb, sx)
        return carry_unused

    lax.fori_loop(0, NCHUNK // 2, step, 0)
    for pe, po, sp in ((pe_a, po_a, sp_a), (pe_b, po_b, sp_b)):
        pltpu.make_async_copy(pe, p_slice(pe_hbm, 0), sp).wait()
        pltpu.make_async_copy(po, p_slice(po_hbm, 0), sp).wait()


_sc_par = pl.kernel(
    _par_body,
    out_type=(jax.ShapeDtypeStruct((B * H,), jnp.float32),
              jax.ShapeDtypeStruct((B * H,), jnp.float32)),
    mesh=plsc.VectorSubcoreMesh(core_axis_name="c", subcore_axis_name="s"),
    compiler_params=pltpu.CompilerParams(needs_layout_passes=False),
    scratch_types=[
        pltpu.VMEM((SPC,), jnp.int32),
        pltpu.VMEM((SPC,), jnp.int32),
        pltpu.VMEM((HPC,), jnp.float32),
        pltpu.VMEM((HPC,), jnp.float32),
        pltpu.VMEM((HPC,), jnp.float32),
        pltpu.VMEM((HPC,), jnp.float32),
        pltpu.SemaphoreType.DMA,
        pltpu.SemaphoreType.DMA,
        pltpu.SemaphoreType.DMA,
        pltpu.SemaphoreType.DMA,
    ],
)

BB = 64


def _tc_body(pe_ref, po_ref, tab_ref, o_ref):
    t0 = tab_ref[0, :]
    diff = tab_ref[1, :] - t0
    he = pe_ref[...][:, :, None] * diff[None, None, :] + t0[None, None, :]
    ho = po_ref[...][:, :, None] * diff[None, None, :] + t0[None, None, :]
    o_ref[...] = jnp.concatenate([he, ho], axis=-1)


_tc_expand = pl.pallas_call(
    _tc_body,
    grid=(B // BB,),
    in_specs=[
        pl.BlockSpec((BB, H), lambda i: (i, 0)),
        pl.BlockSpec((BB, H), lambda i: (i, 0)),
        pl.BlockSpec((2, D), lambda i: (0, 0)),
    ],
    out_specs=pl.BlockSpec((BB, H, 2 * D), lambda i: (i, 0, 0)),
    out_shape=jax.ShapeDtypeStruct((B, H, 2 * D), jnp.float32),
)


def kernel(x, table):
    pe, po = _sc_par(x.reshape(-1))
    out = _tc_expand(pe.reshape(B, H), po.reshape(B, H), table)
    return out.reshape(B, S, D)


# speedup vs baseline: 1.4626x; 1.0066x over previous
import jax
import jax.numpy as jnp
from jax import lax
from jax.experimental import pallas as pl
from jax.experimental.pallas import tpu as pltpu
from jax.experimental.pallas import tpu_sc as plsc

B = 16384
S = 200
D = 64
H = S // 2
NC = 2
NS = 16
NW = NC * NS
RPW = B // NW
CR = 32
NCHUNK = RPW // CR
SPC = CR * S
NG = SPC // 16
HPC = SPC // 2

_DNUMS = lax.GatherDimensionNumbers(
    offset_dims=(), collapsed_slice_dims=(0,), start_index_map=(0,))


def _dg(v, idx):
    return lax.gather(v, idx[:, None], _DNUMS, slice_sizes=(1,),
                      mode=lax.GatherScatterMode.PROMISE_IN_BOUNDS)


def _par_body(x_hbm, pe_hbm, po_hbm, xb_a, xb_b, pe_a, po_a, pe_b, po_b,
              sx_a, sx_b, sp_a, sp_b):
    cid = lax.axis_index("c")
    sid = lax.axis_index("s")
    wid = sid * NC + cid
    tok_base = wid * RPW * S
    lane = lax.iota(jnp.int32, 16)
    zero16 = lane * 0
    even = (lane & 7) * 2
    odd = even + 1

    def x_slice(c):
        off = pl.multiple_of(tok_base + c * SPC, 32)
        return x_hbm.at[pl.ds(off, SPC)]

    def p_slice(hbm, c):
        off = pl.multiple_of((tok_base + c * SPC) // 2, 32)
        return hbm.at[pl.ds(off, HPC)]

    def compute_chunk(xb, pe, po):
        def scan_group(g, carry):
            v = xb[pl.ds(g * 16, 16)]
            scan = plsc.cumsum(v)
            t = scan + carry
            ex = t - v
            gm = g % 25
            lam = jnp.where(gm == 0, 0, jnp.where(gm == 12, 8, 16))
            sub = _dg(ex, zero16 + jnp.minimum(lam, 15))
            tot = jnp.where(lane < lam, t, t - sub)
            return tot & 1, _dg(tot, zero16 + 15)

        def gpair(g2, carry):
            pa, carry = scan_group(2 * g2, carry)
            pb, carry = scan_group(2 * g2 + 1, carry)
            ev = jnp.where(lane < 8, _dg(pa, even), _dg(pb, even))
            od = jnp.where(lane < 8, _dg(pa, odd), _dg(pb, odd))
            pe[pl.ds(g2 * 16, 16)] = ev.astype(jnp.float32)
            po[pl.ds(g2 * 16, 16)] = od.astype(jnp.float32)
            return carry

        lax.fori_loop(0, NG // 2, gpair, jnp.zeros((16,), jnp.int32))

    pltpu.async_copy(x_slice(0), xb_a, sx_a)
    pltpu.async_copy(x_slice(1), xb_b, sx_b)

    bufs = ((xb_a, pe_a, po_a, sx_a, sp_a), (xb_b, pe_b, po_b, sx_b, sp_b))

    def step(i, carry_unused):
        for j, (xb, pe, po, sx, sp) in enumerate(bufs):
            c = 2 * i + j
            pltpu.make_async_copy(x_slice(0), xb, sx).wait()

            @pl.when(i > 0)
            def _():
                pltpu.make_async_copy(pe, p_slice(pe_hbm, 0), sp).wait()
                pltpu.make_async_copy(po, p_slice(po_hbm, 0), sp).wait()

            compute_chunk(xb, pe, po)
            pltpu.async_copy(pe, p_slice(pe_hbm, c), sp)
            pltpu.async_copy(po, p_slice(po_hbm, c), sp)

            @pl.when(c + 2 < NCHUNK)
            def _():
                pltpu.async_copy(x_slice(c + 2), xb, sx)
        return carry_unused

    lax.fori_loop(0, NCHUNK // 2, step, 0)
    for pe, po, sp in ((pe_a, po_a, sp_a), (pe_b, po_b, sp_b)):
        pltpu.make_async_copy(pe, p_slice(pe_hbm, 0), sp).wait()
        pltpu.make_async_copy(po, p_slice(po_hbm, 0), sp).wait()


_sc_par = pl.kernel(
    _par_body,
    out_type=(jax.ShapeDtypeStruct((B * H,), jnp.float32),
              jax.ShapeDtypeStruct((B * H,), jnp.float32)),
    mesh=plsc.VectorSubcoreMesh(core_axis_name="c", subcore_axis_name="s"),
    compiler_params=pltpu.CompilerParams(needs_layout_passes=False),
    scratch_types=[
        pltpu.VMEM((SPC,), jnp.int32),
        pltpu.VMEM((SPC,), jnp.int32),
        pltpu.VMEM((HPC,), jnp.float32),
        pltpu.VMEM((HPC,), jnp.float32),
        pltpu.VMEM((HPC,), jnp.float32),
        pltpu.VMEM((HPC,), jnp.float32),
        pltpu.SemaphoreType.DMA,
        pltpu.SemaphoreType.DMA,
        pltpu.SemaphoreType.DMA,
        pltpu.SemaphoreType.DMA,
    ],
)

BB = 128


def _tc_body(pe_ref, po_ref, tab_ref, o_ref):
    t0 = tab_ref[0, :]
    diff = tab_ref[1, :] - t0
    he = pe_ref[...][:, :, None] * diff[None, None, :] + t0[None, None, :]
    ho = po_ref[...][:, :, None] * diff[None, None, :] + t0[None, None, :]
    o_ref[...] = jnp.concatenate([he, ho], axis=-1)


_tc_expand = pl.pallas_call(
    _tc_body,
    grid=(B // BB,),
    in_specs=[
        pl.BlockSpec((BB, H), lambda i: (i, 0)),
        pl.BlockSpec((BB, H), lambda i: (i, 0)),
        pl.BlockSpec((2, D), lambda i: (0, 0)),
    ],
    out_specs=pl.BlockSpec((BB, H, 2 * D), lambda i: (i, 0, 0)),
    out_shape=jax.ShapeDtypeStruct((B, H, 2 * D), jnp.float32),
)


def kernel(x, table):
    pe, po = _sc_par(x.reshape(-1))
    out = _tc_expand(pe.reshape(B, H), po.reshape(B, H), table)
    return out.reshape(B, S, D)
